# baseline (speedup 1.0000x reference)
import functools

import jax
import jax.numpy as jnp
from jax import lax
from jax.experimental import pallas as pl
from jax.experimental.pallas import tpu as pltpu
from jax.experimental.pallas import tpu_sc as plsc

D = 64
SCALE = 8.0


@functools.lru_cache(maxsize=None)
def _make_kernel(B: int, C: int):
    info = plsc.get_sparse_core_info()
    NC, NS = info.num_cores, info.num_subcores
    NW = NC * NS
    assert B % NW == 0
    b_per_w = B // NW
    assert b_per_w % C == 0 and C % 8 == 0
    n_chunks = b_per_w // C
    mesh = plsc.VectorSubcoreMesh(core_axis_name="c", subcore_axis_name="s")

    @functools.partial(
        pl.kernel,
        mesh=mesh,
        out_type=jax.ShapeDtypeStruct((B, D), jnp.float32),
        scratch_types=[
            pltpu.VMEM((C,), jnp.int32),
            pltpu.VMEM((C, D), jnp.float32),
            pltpu.SemaphoreType.DMA,
        ],
        compiler_params=pltpu.CompilerParams(use_tc_tiling_on_sc=False),
    )
    def k(idx_hbm, table_hbm, out_hbm, idx_v, rows_v, sem):
        wid = lax.axis_index("s") * NC + lax.axis_index("c")
        base = wid * b_per_w

        def chunk_body(c, _):
            off = base + c * C
            pltpu.sync_copy(idx_hbm.at[pl.ds(off, C)], idx_v)
            pltpu.async_copy(table_hbm.at[idx_v], rows_v, sem).wait()

            def scale_row(r, _):
                for j in range(D // 16):
                    sl = pl.ds(j * 16, 16)
                    rows_v[r, sl] = rows_v[r, sl] * SCALE
                return 0

            lax.fori_loop(0, C, scale_row, 0)
            pltpu.sync_copy(rows_v, out_hbm.at[pl.ds(off, C)])
            return 0

        lax.fori_loop(0, n_chunks, chunk_body, 0)

    return k


def kernel(x, table):
    B = x.shape[0] * x.shape[1]
    flat = x.reshape(B).astype(jnp.int32)
    out = _make_kernel(B, 800)(flat, table)
    return out.reshape(x.shape[0], x.shape[1], D)


# speedup vs baseline: 1.0002x; 1.0002x over previous
import functools

import jax
import jax.numpy as jnp
from jax import lax
from jax.experimental import pallas as pl
from jax.experimental.pallas import tpu as pltpu
from jax.experimental.pallas import tpu_sc as plsc

D = 64
SCALE = 8.0


@functools.lru_cache(maxsize=None)
def _make_kernel(NB: int, S: int, R: int):
    info = plsc.get_sparse_core_info()
    NC, NS = info.num_cores, info.num_subcores
    NW = NC * NS
    assert NB % NW == 0
    rows_per_w = NB // NW
    assert rows_per_w % R == 0
    n_chunks = rows_per_w // R
    mesh = plsc.VectorSubcoreMesh(core_axis_name="c", subcore_axis_name="s")

    @functools.partial(
        pl.kernel,
        mesh=mesh,
        out_type=jax.ShapeDtypeStruct((NB, S, D), jnp.float32),
        scratch_types=[
            pltpu.VMEM((R, S), jnp.int32),
            pltpu.VMEM((R, S, D), jnp.float32),
            pltpu.SemaphoreType.DMA,
        ],
        compiler_params=pltpu.CompilerParams(use_tc_tiling_on_sc=False),
    )
    def k(idx_hbm, table_hbm, out_hbm, idx_v, rows_v, sem):
        wid = lax.axis_index("s") * NC + lax.axis_index("c")
        base = wid * rows_per_w

        def chunk_body(c, _):
            off = base + c * R
            pltpu.sync_copy(idx_hbm.at[pl.ds(off, R)], idx_v)
            for a in range(R):
                pltpu.async_copy(
                    table_hbm.at[idx_v.at[a]], rows_v.at[a], sem
                ).wait()

            def scale_row(r, _):
                for j in range(D // 16):
                    sl = pl.ds(j * 16, 16)
                    for a in range(R):
                        rows_v[a, r, sl] = rows_v[a, r, sl] * SCALE
                return 0

            lax.fori_loop(0, S, scale_row, 0)
            pltpu.sync_copy(rows_v, out_hbm.at[pl.ds(off, R)])
            return 0

        lax.fori_loop(0, n_chunks, chunk_body, 0)

    return k


def kernel(x, table):
    NB, S = x.shape
    out = _make_kernel(NB, S, 4)(x.astype(jnp.int32), table)
    return out


# speedup vs baseline: 1.2946x; 1.2942x over previous
import functools

import jax
import jax.numpy as jnp
from jax import lax
from jax.experimental import pallas as pl
from jax.experimental.pallas import tpu as pltpu
from jax.experimental.pallas import tpu_sc as plsc

D = 64
SCALE = 8.0


@functools.lru_cache(maxsize=None)
def _make_kernel(NB: int, S: int, R: int):
    info = plsc.get_sparse_core_info()
    NC, NS = info.num_cores, info.num_subcores
    NW = NC * NS
    assert NB % NW == 0
    rows_per_w = NB // NW
    assert rows_per_w % R == 0
    n_chunks = rows_per_w // R
    C = R * S
    assert C % 16 == 0
    mesh = plsc.VectorSubcoreMesh(core_axis_name="c", subcore_axis_name="s")

    @functools.partial(
        pl.kernel,
        mesh=mesh,
        out_type=jax.ShapeDtypeStruct((NB, S, D), jnp.float32),
        scratch_types=[
            pltpu.VMEM((C,), jnp.int32),
            pltpu.VMEM((C, D), jnp.float32),
            pltpu.SemaphoreType.DMA,
        ],
        compiler_params=pltpu.CompilerParams(use_tc_tiling_on_sc=True),
    )
    def k(idx_hbm, table_hbm, out_hbm, idx_v, rows_v, gsem):
        wid = lax.axis_index("s") * NC + lax.axis_index("c")
        base_row = wid * rows_per_w

        def chunk_body(cidx, _):
            b0 = base_row + cidx * R
            off = b0 * S
            pltpu.sync_copy(idx_hbm.at[pl.ds(off, C)], idx_v)

            def fire(blk, _):
                k0 = blk * 16
                v = idx_v[pl.ds(k0, 16)]
                for i in range(16):
                    pltpu.async_copy(
                        table_hbm.at[v[i]], rows_v.at[k0 + i], gsem
                    )
                return 0

            lax.fori_loop(0, C // 16, fire, 0)

            for a in range(R):
                pltpu.make_async_copy(
                    out_hbm.at[b0 + a], rows_v.at[pl.ds(a * S, S)], gsem
                ).wait()

            def scale_row(s, _):
                for j in range(D // 16):
                    sl = pl.ds(j * 16, 16)
                    rows_v[s, sl] = rows_v[s, sl] * SCALE
                return 0

            lax.fori_loop(0, C, scale_row, 0)
            for a in range(R):
                pltpu.sync_copy(
                    rows_v.at[pl.ds(a * S, S)], out_hbm.at[b0 + a]
                )
            return 0

        lax.fori_loop(0, n_chunks, chunk_body, 0)

    return k


def kernel(x, table):
    NB, S = x.shape
    flat = x.reshape(NB * S).astype(jnp.int32)
    out = _make_kernel(NB, S, 4)(flat, table)
    return out
